# Initial kernel scaffold; baseline (speedup 1.0000x reference)
import jax
import jax.numpy as jnp
from jax.experimental import pallas as pl


def kernel(x, hidden_weights, out_weights, bias, hidden_idx, out_idx):
    raise NotImplementedError("write your pallas kernel here")


# speedup vs baseline: 3.3626x; 3.3626x over previous
import functools

import jax
import jax.numpy as jnp
from jax import lax
from jax.experimental import pallas as pl
from jax.experimental.pallas import tpu as pltpu
from jax.experimental.pallas import tpu_sc as plsc

NUM_INPUT = 4096
HIDDEN_BATCHES = 23
HIDDEN_SIZE = 4096
NUM_OUTPUT = 1024
FAN_IN = 32
LANES = 16
NUM_TILES = 16
ROWS_HID = HIDDEN_SIZE // NUM_TILES
ROWS_OUT = NUM_OUTPUT // NUM_TILES


def _fan_in_rows(wbuf, ibuf, vals, bias_vec, row0, pstart):
    lane = lax.iota(jnp.int32, LANES)
    vbase = row0 * FAN_IN + lane * FAN_IN
    pvec = jnp.full((LANES,), pstart, dtype=jnp.int32)
    accs = [bias_vec, jnp.zeros((LANES,), jnp.float32),
            jnp.zeros((LANES,), jnp.float32), jnp.zeros((LANES,), jnp.float32)]
    for f in range(FAN_IN):
        fidx = vbase + f
        w = plsc.load_gather(wbuf, [fidx])
        gi = plsc.load_gather(ibuf, [fidx]) - pvec
        v = plsc.load_gather(vals, [gi])
        accs[f % 4] = accs[f % 4] + w * v
    return (accs[0] + accs[1]) + (accs[2] + accs[3])


def _body(x_hbm, hw_hbm, ow_hbm, bias_hbm, hi_hbm, oi_hbm, out_hbm,
          vals, wbuf, ibuf, bbuf, obuf, shared):
    cid = lax.axis_index("c")
    sid = lax.axis_index("s")

    @pl.when(cid == 0)
    def _():
        base = sid * ROWS_HID
        pltpu.sync_copy(x_hbm, vals)

        def layer(t, carry):
            pltpu.sync_copy(hw_hbm.at[t, pl.ds(base * FAN_IN, ROWS_HID * FAN_IN)], wbuf)
            pltpu.sync_copy(hi_hbm.at[t, pl.ds(base * FAN_IN, ROWS_HID * FAN_IN)], ibuf)
            pltpu.sync_copy(bias_hbm.at[pl.ds(t * HIDDEN_SIZE + base, ROWS_HID)], bbuf)
            pstart = t * HIDDEN_SIZE

            def rows(r, c2):
                row0 = r * LANES
                bv = bbuf[pl.ds(row0, LANES)]
                a = _fan_in_rows(wbuf, ibuf, vals, bv, row0, pstart)
                obuf[pl.ds(row0, LANES)] = a / (1.0 + jnp.exp(-a))
                return c2

            lax.fori_loop(0, ROWS_HID // LANES, rows, 0)

            slot = lax.rem(t, 2)
            pltpu.sync_copy(obuf, shared.at[slot, pl.ds(base, ROWS_HID)])
            plsc.subcore_barrier()
            pltpu.sync_copy(shared.at[slot], vals)
            return carry

        lax.fori_loop(0, HIDDEN_BATCHES, layer, 0)

        base_o = sid * ROWS_OUT
        pltpu.sync_copy(ow_hbm.at[pl.ds(base_o * FAN_IN, ROWS_OUT * FAN_IN)],
                        wbuf.at[pl.ds(0, ROWS_OUT * FAN_IN)])
        pltpu.sync_copy(oi_hbm.at[pl.ds(base_o * FAN_IN, ROWS_OUT * FAN_IN)],
                        ibuf.at[pl.ds(0, ROWS_OUT * FAN_IN)])
        pltpu.sync_copy(
            bias_hbm.at[pl.ds(HIDDEN_BATCHES * HIDDEN_SIZE + base_o, ROWS_OUT)],
            bbuf.at[pl.ds(0, ROWS_OUT)])
        pstart_o = HIDDEN_BATCHES * HIDDEN_SIZE

        def out_rows(r, c2):
            row0 = r * LANES
            bv = bbuf[pl.ds(row0, LANES)]
            a = _fan_in_rows(wbuf, ibuf, vals, bv, row0, pstart_o)
            obuf[pl.ds(row0, LANES)] = a
            return c2

        lax.fori_loop(0, ROWS_OUT // LANES, out_rows, 0)
        pltpu.sync_copy(obuf.at[pl.ds(0, ROWS_OUT)], out_hbm.at[pl.ds(base_o, ROWS_OUT)])


def kernel(x, hidden_weights, out_weights, bias, hidden_idx, out_idx):
    hw = hidden_weights.reshape(HIDDEN_BATCHES, HIDDEN_SIZE * FAN_IN)
    hi = hidden_idx.reshape(HIDDEN_BATCHES, HIDDEN_SIZE * FAN_IN)
    ow = out_weights.reshape(NUM_OUTPUT * FAN_IN)
    oi = out_idx.reshape(NUM_OUTPUT * FAN_IN)

    mesh = plsc.VectorSubcoreMesh(core_axis_name="c", subcore_axis_name="s")
    run = pl.kernel(
        _body,
        mesh=mesh,
        compiler_params=pltpu.CompilerParams(
            use_tc_tiling_on_sc=False, needs_layout_passes=False),
        out_type=jax.ShapeDtypeStruct((NUM_OUTPUT,), jnp.float32),
        scratch_types=[
            pltpu.VMEM((HIDDEN_SIZE,), jnp.float32),
            pltpu.VMEM((ROWS_HID * FAN_IN,), jnp.float32),
            pltpu.VMEM((ROWS_HID * FAN_IN,), jnp.int32),
            pltpu.VMEM((ROWS_HID,), jnp.float32),
            pltpu.VMEM((ROWS_HID,), jnp.float32),
            pltpu.VMEM_SHARED((2, HIDDEN_SIZE), jnp.float32),
        ],
    )
    return run(x, hw, ow, bias, hi, oi)


# speedup vs baseline: 6.8031x; 2.0232x over previous
import functools

import jax
import jax.numpy as jnp
from jax import lax
from jax.experimental import pallas as pl
from jax.experimental.pallas import tpu as pltpu
from jax.experimental.pallas import tpu_sc as plsc

NUM_INPUT = 4096
HIDDEN_BATCHES = 23
HIDDEN_SIZE = 4096
NUM_OUTPUT = 1024
FAN_IN = 32
LANES = 16
NUM_TILES = 16
ROWS_HID = HIDDEN_SIZE // NUM_TILES
ROWS_OUT = NUM_OUTPUT // NUM_TILES


def _fan_in_rows(wbuf, ibuf, vals, bias_vec, row0, pstart):
    pvec = jnp.full((LANES,), pstart, dtype=jnp.int32)
    accs = [bias_vec, jnp.zeros((LANES,), jnp.float32),
            jnp.zeros((LANES,), jnp.float32), jnp.zeros((LANES,), jnp.float32)]
    for f in range(FAN_IN):
        w = wbuf[f, pl.ds(row0, LANES)]
        gi = ibuf[f, pl.ds(row0, LANES)] - pvec
        v = plsc.load_gather(vals, [gi])
        accs[f % 4] = accs[f % 4] + w * v
    return (accs[0] + accs[1]) + (accs[2] + accs[3])


def _body(x_hbm, hw_hbm, ow_hbm, bias_hbm, hi_hbm, oi_hbm, out_hbm,
          vals, wbuf, ibuf, owbuf, oibuf, bbuf, obuf, shared):
    cid = lax.axis_index("c")
    sid = lax.axis_index("s")

    @pl.when(cid == 0)
    def _():
        base = sid * ROWS_HID
        pltpu.sync_copy(x_hbm, vals)

        def layer(t, carry):
            pltpu.sync_copy(hw_hbm.at[t, sid], wbuf)
            pltpu.sync_copy(hi_hbm.at[t, sid], ibuf)
            pltpu.sync_copy(bias_hbm.at[pl.ds(t * HIDDEN_SIZE + base, ROWS_HID)], bbuf)
            pstart = t * HIDDEN_SIZE

            def rows(r, c2):
                row0 = r * LANES
                bv = bbuf[pl.ds(row0, LANES)]
                a = _fan_in_rows(wbuf, ibuf, vals, bv, row0, pstart)
                obuf[pl.ds(row0, LANES)] = a / (1.0 + jnp.exp(-a))
                return c2

            lax.fori_loop(0, ROWS_HID // LANES, rows, 0)

            slot = lax.rem(t, 2)
            pltpu.sync_copy(obuf, shared.at[slot, pl.ds(base, ROWS_HID)])
            plsc.subcore_barrier()
            pltpu.sync_copy(shared.at[slot], vals)
            return carry

        lax.fori_loop(0, HIDDEN_BATCHES, layer, 0)

        base_o = sid * ROWS_OUT
        pltpu.sync_copy(ow_hbm.at[sid], owbuf)
        pltpu.sync_copy(oi_hbm.at[sid], oibuf)
        pltpu.sync_copy(
            bias_hbm.at[pl.ds(HIDDEN_BATCHES * HIDDEN_SIZE + base_o, ROWS_OUT)],
            bbuf.at[pl.ds(0, ROWS_OUT)])
        pstart_o = HIDDEN_BATCHES * HIDDEN_SIZE

        def out_rows(r, c2):
            row0 = r * LANES
            bv = bbuf[pl.ds(row0, LANES)]
            a = _fan_in_rows(owbuf, oibuf, vals, bv, row0, pstart_o)
            obuf[pl.ds(row0, LANES)] = a
            return c2

        lax.fori_loop(0, ROWS_OUT // LANES, out_rows, 0)
        pltpu.sync_copy(obuf.at[pl.ds(0, ROWS_OUT)], out_hbm.at[pl.ds(base_o, ROWS_OUT)])


def kernel(x, hidden_weights, out_weights, bias, hidden_idx, out_idx):
    hw = hidden_weights.reshape(
        HIDDEN_BATCHES, NUM_TILES, ROWS_HID, FAN_IN).transpose(0, 1, 3, 2)
    hi = hidden_idx.reshape(
        HIDDEN_BATCHES, NUM_TILES, ROWS_HID, FAN_IN).transpose(0, 1, 3, 2)
    ow = out_weights.reshape(NUM_TILES, ROWS_OUT, FAN_IN).transpose(0, 2, 1)
    oi = out_idx.reshape(NUM_TILES, ROWS_OUT, FAN_IN).transpose(0, 2, 1)

    mesh = plsc.VectorSubcoreMesh(core_axis_name="c", subcore_axis_name="s")
    run = pl.kernel(
        _body,
        mesh=mesh,
        compiler_params=pltpu.CompilerParams(
            use_tc_tiling_on_sc=False, needs_layout_passes=False),
        out_type=jax.ShapeDtypeStruct((NUM_OUTPUT,), jnp.float32),
        scratch_types=[
            pltpu.VMEM((HIDDEN_SIZE,), jnp.float32),
            pltpu.VMEM((FAN_IN, ROWS_HID), jnp.float32),
            pltpu.VMEM((FAN_IN, ROWS_HID), jnp.int32),
            pltpu.VMEM((FAN_IN, ROWS_OUT), jnp.float32),
            pltpu.VMEM((FAN_IN, ROWS_OUT), jnp.int32),
            pltpu.VMEM((ROWS_HID,), jnp.float32),
            pltpu.VMEM((ROWS_HID,), jnp.float32),
            pltpu.VMEM_SHARED((2, HIDDEN_SIZE), jnp.float32),
        ],
    )
    return run(x, hw, ow, bias, hi, oi)


# speedup vs baseline: 6.8167x; 1.0020x over previous
import functools

import jax
import jax.numpy as jnp
from jax import lax
from jax.experimental import pallas as pl
from jax.experimental.pallas import tpu as pltpu
from jax.experimental.pallas import tpu_sc as plsc

NUM_INPUT = 4096
HIDDEN_BATCHES = 23
HIDDEN_SIZE = 4096
NUM_OUTPUT = 1024
FAN_IN = 32
LANES = 16
NUM_TILES = 16
ROWS_HID = HIDDEN_SIZE // NUM_TILES
ROWS_OUT = NUM_OUTPUT // NUM_TILES


def _fan_in_rows(wbuf, ibuf, vals, bias_vec, row0, pstart, ncols):
    pvec = jnp.full((LANES,), pstart, dtype=jnp.int32)
    accs = [bias_vec, jnp.zeros((LANES,), jnp.float32),
            jnp.zeros((LANES,), jnp.float32), jnp.zeros((LANES,), jnp.float32)]
    for f in range(FAN_IN):
        w = wbuf[pl.ds(f * ncols + row0, LANES)]
        gi = ibuf[pl.ds(f * ncols + row0, LANES)] - pvec
        v = plsc.load_gather(vals, [gi])
        accs[f % 4] = accs[f % 4] + w * v
    return (accs[0] + accs[1]) + (accs[2] + accs[3])


def _body(x_hbm, hw_hbm, ow_hbm, bias_hbm, hi_hbm, oi_hbm, out_hbm,
          vals, wbuf, ibuf, owbuf, oibuf, bbuf, obuf, shared):
    cid = lax.axis_index("c")
    sid = lax.axis_index("s")

    @pl.when(cid == 0)
    def _():
        base = sid * ROWS_HID
        pltpu.sync_copy(x_hbm, vals)

        chunk = ROWS_HID * FAN_IN

        def layer(t, carry):
            off = t * (HIDDEN_SIZE * FAN_IN) + sid * chunk
            pltpu.sync_copy(hw_hbm.at[pl.ds(off, chunk)], wbuf)
            pltpu.sync_copy(hi_hbm.at[pl.ds(off, chunk)], ibuf)
            pltpu.sync_copy(bias_hbm.at[pl.ds(t * HIDDEN_SIZE + base, ROWS_HID)], bbuf)
            pstart = t * HIDDEN_SIZE

            def rows(r, c2):
                row0 = r * LANES
                bv = bbuf[pl.ds(row0, LANES)]
                a = _fan_in_rows(wbuf, ibuf, vals, bv, row0, pstart, ROWS_HID)
                obuf[pl.ds(row0, LANES)] = a / (1.0 + jnp.exp(-a))
                return c2

            lax.fori_loop(0, ROWS_HID // LANES, rows, 0)

            slot = lax.rem(t, 2)
            pltpu.sync_copy(obuf, shared.at[slot, pl.ds(base, ROWS_HID)])
            plsc.subcore_barrier()
            pltpu.sync_copy(shared.at[slot], vals)
            return carry

        lax.fori_loop(0, HIDDEN_BATCHES, layer, 0)

        base_o = sid * ROWS_OUT
        ochunk = ROWS_OUT * FAN_IN
        pltpu.sync_copy(ow_hbm.at[pl.ds(sid * ochunk, ochunk)], owbuf)
        pltpu.sync_copy(oi_hbm.at[pl.ds(sid * ochunk, ochunk)], oibuf)
        pltpu.sync_copy(
            bias_hbm.at[pl.ds(HIDDEN_BATCHES * HIDDEN_SIZE + base_o, ROWS_OUT)],
            bbuf.at[pl.ds(0, ROWS_OUT)])
        pstart_o = HIDDEN_BATCHES * HIDDEN_SIZE

        def out_rows(r, c2):
            row0 = r * LANES
            bv = bbuf[pl.ds(row0, LANES)]
            a = _fan_in_rows(owbuf, oibuf, vals, bv, row0, pstart_o, ROWS_OUT)
            obuf[pl.ds(row0, LANES)] = a
            return c2

        lax.fori_loop(0, ROWS_OUT // LANES, out_rows, 0)
        pltpu.sync_copy(obuf.at[pl.ds(0, ROWS_OUT)], out_hbm.at[pl.ds(base_o, ROWS_OUT)])


def kernel(x, hidden_weights, out_weights, bias, hidden_idx, out_idx):
    hw = hidden_weights.reshape(
        HIDDEN_BATCHES, NUM_TILES, ROWS_HID, FAN_IN).transpose(0, 1, 3, 2).reshape(-1)
    hi = hidden_idx.reshape(
        HIDDEN_BATCHES, NUM_TILES, ROWS_HID, FAN_IN).transpose(0, 1, 3, 2).reshape(-1)
    ow = out_weights.reshape(
        NUM_TILES, ROWS_OUT, FAN_IN).transpose(0, 2, 1).reshape(-1)
    oi = out_idx.reshape(
        NUM_TILES, ROWS_OUT, FAN_IN).transpose(0, 2, 1).reshape(-1)

    mesh = plsc.VectorSubcoreMesh(core_axis_name="c", subcore_axis_name="s")
    run = pl.kernel(
        _body,
        mesh=mesh,
        compiler_params=pltpu.CompilerParams(
            use_tc_tiling_on_sc=False, needs_layout_passes=False),
        out_type=jax.ShapeDtypeStruct((NUM_OUTPUT,), jnp.float32),
        scratch_types=[
            pltpu.VMEM((HIDDEN_SIZE,), jnp.float32),
            pltpu.VMEM((FAN_IN * ROWS_HID,), jnp.float32),
            pltpu.VMEM((FAN_IN * ROWS_HID,), jnp.int32),
            pltpu.VMEM((FAN_IN * ROWS_OUT,), jnp.float32),
            pltpu.VMEM((FAN_IN * ROWS_OUT,), jnp.int32),
            pltpu.VMEM((ROWS_HID,), jnp.float32),
            pltpu.VMEM((ROWS_HID,), jnp.float32),
            pltpu.VMEM_SHARED((2, HIDDEN_SIZE), jnp.float32),
        ],
    )
    return run(x, hw, ow, bias, hi, oi)
